# baseline (speedup 1.0000x reference)
import functools

import jax
import jax.numpy as jnp
from jax import lax
from jax.experimental import pallas as pl
from jax.experimental.pallas import tpu as pltpu
from jax.experimental.pallas import tpu_sc as plsc

B = 1024
F = 26
V = 100000
D = 32
HIDDEN = 1024
OUT = 1

NC = 2
NS = 16
NW = NC * NS
ROWS = B * F
RPW = ROWS // NW
CHUNKS = 8
CPW = RPW // CHUNKS


def _gather_body(tab_hbm, idx_hbm, out_hbm, idx_v, rows_v, sem):
    wid = lax.axis_index("s") * NC + lax.axis_index("c")
    pltpu.sync_copy(idx_hbm.at[wid], idx_v)
    copies = [
        pltpu.async_copy(tab_hbm.at[idx_v.at[j]], rows_v.at[j], sem)
        for j in range(CHUNKS)
    ]
    for c in copies:
        c.wait()
    pltpu.sync_copy(rows_v, out_hbm.at[wid])


_sc_gather = functools.partial(
    pl.kernel,
    out_type=jax.ShapeDtypeStruct((NW, CHUNKS, CPW, D), jnp.float32),
    scratch_types=[
        pltpu.VMEM((CHUNKS, CPW), jnp.int32),
        pltpu.VMEM((CHUNKS, CPW, D), jnp.float32),
        pltpu.SemaphoreType.DMA,
    ],
    mesh=plsc.VectorSubcoreMesh(core_axis_name="c", subcore_axis_name="s"),
    compiler_params=pltpu.CompilerParams(use_tc_tiling_on_sc=False),
)(_gather_body)


def _mlp_body(emb_ref, w1_ref, b1_ref, w2_ref, b2_ref, out_ref):
    hid = jnp.dot(emb_ref[...], w1_ref[...],
                  preferred_element_type=jnp.float32) + b1_ref[...]
    rel = jnp.maximum(hid, 0.0)
    out_ref[...] = jnp.dot(rel, w2_ref[...],
                           preferred_element_type=jnp.float32) + b2_ref[...]


def kernel(features, tables, W1, b1, W2, b2):
    tab_flat = tables.reshape(F * V, D)
    offsets = (jnp.arange(F, dtype=jnp.int32) * V)[None, :]
    flat_idx = (features + offsets).reshape(NW, CHUNKS, CPW)
    emb = _sc_gather(tab_flat, flat_idx).reshape(B, F * D)
    out = pl.pallas_call(
        _mlp_body,
        out_shape=jax.ShapeDtypeStruct((B, OUT), jnp.float32),
    )(emb, W1, b1[None, :], W2, b2[None, :])
    return out
